# Initial kernel scaffold; baseline (speedup 1.0000x reference)
import jax
import jax.numpy as jnp
from jax.experimental import pallas as pl


def kernel(features, adj, adj2, W1a, W1b, W2a, W2b, w_omega, b_omega, u_omega, clusters, W_rec, b_rec, W_out, b_out):
    raise NotImplementedError("write your pallas kernel here")


# speedup vs baseline: 1.2692x; 1.2692x over previous
import jax
import jax.numpy as jnp
from jax import lax
from jax.experimental import pallas as pl
from jax.experimental.pallas import tpu as pltpu

_N = 3025
_DIN = 1870
_H1 = 32
_H2 = 16

_BN1 = 512
_BN2 = 256
_BND = 256


def _f32dot(a, b):
    return lax.dot(a, b, preferred_element_type=jnp.float32)


def _bf16dot(a, b):
    return lax.dot(a.astype(jnp.bfloat16), b.astype(jnp.bfloat16),
                   preferred_element_type=jnp.float32)


def _xw_body(x_ref, w_ref, o_ref):
    o_ref[...] = _bf16dot(x_ref[...], w_ref[...])


def _xw_call(features, wcat):
    grid = (pl.cdiv(_N, _BN1),)
    return pl.pallas_call(
        _xw_body,
        grid=grid,
        in_specs=[
            pl.BlockSpec((_BN1, _DIN), lambda i: (i, 0)),
            pl.BlockSpec((_DIN, 2 * _H1), lambda i: (0, 0)),
        ],
        out_specs=pl.BlockSpec((_BN1, 2 * _H1), lambda i: (i, 0)),
        out_shape=jax.ShapeDtypeStruct((_N, 2 * _H1), jnp.float32),
    )(features, wcat)


def _layer1_body(adj_ref, adj2_ref, xw_ref, w2_ref, p_ref):
    xw = xw_ref[...].astype(jnp.bfloat16)
    a1 = adj_ref[...].astype(jnp.bfloat16)
    a2 = adj2_ref[...].astype(jnp.bfloat16)
    h1 = jnp.maximum(lax.dot(a1, xw[:, :_H1], preferred_element_type=jnp.float32), 0.0)
    h2 = jnp.maximum(lax.dot(a2, xw[:, _H1:], preferred_element_type=jnp.float32), 0.0)
    w2 = w2_ref[...]
    p1 = _f32dot(h1, w2[:, :_H2])
    p2 = _f32dot(h2, w2[:, _H2:])
    p_ref[...] = jnp.concatenate([p1, p2], axis=1)


def _layer1_call(adj, adj2, xw, w2cat):
    grid = (pl.cdiv(_N, _BN2),)
    return pl.pallas_call(
        _layer1_body,
        grid=grid,
        in_specs=[
            pl.BlockSpec((_BN2, _N), lambda i: (i, 0)),
            pl.BlockSpec((_BN2, _N), lambda i: (i, 0)),
            pl.BlockSpec((_N, 2 * _H1), lambda i: (0, 0)),
            pl.BlockSpec((_H1, 2 * _H2), lambda i: (0, 0)),
        ],
        out_specs=pl.BlockSpec((_BN2, 2 * _H2), lambda i: (i, 0)),
        out_shape=jax.ShapeDtypeStruct((_N, 2 * _H2), jnp.float32),
    )(adj, adj2, xw, w2cat)


def _layer2_body(adj_ref, adj2_ref, p_ref, womega_ref, bomega_ref,
                 uomega_ref, clusters_ref, wrec_ref, brec_ref, wout_ref,
                 bout_ref,
                 emb_ref, al_ref, q_ref, z1_ref, z2_ref, pos_ref, rec3_ref):
    p = p_ref[...].astype(jnp.bfloat16)
    a1 = adj_ref[...].astype(jnp.bfloat16)
    a2 = adj2_ref[...].astype(jnp.bfloat16)
    e1 = lax.dot(a1, p[:, :_H2], preferred_element_type=jnp.float32)
    e2 = lax.dot(a2, p[:, _H2:], preferred_element_type=jnp.float32)

    w_om = womega_ref[...]
    b_om = bomega_ref[...][None, :]
    u_om = uomega_ref[...][None, :]
    v1 = jnp.tanh(_f32dot(e1, w_om) + b_om)
    v2 = jnp.tanh(_f32dot(e2, w_om) + b_om)
    vu1 = jnp.sum(v1 * u_om, axis=1, keepdims=True)
    vu2 = jnp.sum(v2 * u_om, axis=1, keepdims=True)
    m = jnp.maximum(vu1, vu2)
    x1 = jnp.exp(vu1 - m)
    x2 = jnp.exp(vu2 - m)
    s = x1 + x2
    al1 = x1 / s
    al2 = x2 / s
    emb = al1 * e1 + al2 * e2
    emb_ref[...] = emb
    al_ref[...] = jnp.concatenate([al1, al2], axis=1)

    c = clusters_ref[...]
    qs = []
    for k in range(3):
        ck = c[k:k + 1, :]
        d2 = jnp.sum((emb - ck) ** 2, axis=1, keepdims=True)
        qs.append(1.0 / (1.0 + d2))
    qcat = jnp.concatenate(qs, axis=1)
    q_ref[...] = qcat / jnp.sum(qcat, axis=1, keepdims=True)

    n1 = jnp.sqrt(jnp.sum(e1 * e1, axis=1, keepdims=True)) + 1e-8
    n2 = jnp.sqrt(jnp.sum(e2 * e2, axis=1, keepdims=True)) + 1e-8
    z1 = e1 / n1
    z2 = e2 / n2
    z1_ref[...] = z1
    z2_ref[...] = z2
    pos_ref[...] = jnp.sum(z1 * z2, axis=1, keepdims=True)

    r = jnp.maximum(_f32dot(emb, wrec_ref[...]) + brec_ref[...][None, :], 0.0)
    rec3_ref[...] = _bf16dot(r, wout_ref[...]) + bout_ref[...][None, :]


def _layer2_call(adj, adj2, p, w_omega, b_omega, u_omega, clusters,
                 w_rec, b_rec, w_out, b_out):
    grid = (pl.cdiv(_N, _BN2),)
    row = lambda i: (i, 0)
    rep = lambda i: (0, 0)
    rep1 = lambda i: (0,)
    return pl.pallas_call(
        _layer2_body,
        grid=grid,
        in_specs=[
            pl.BlockSpec((_BN2, _N), row),
            pl.BlockSpec((_BN2, _N), row),
            pl.BlockSpec((_N, 2 * _H2), rep),
            pl.BlockSpec((_H2, 16), rep),
            pl.BlockSpec((16,), rep1),
            pl.BlockSpec((16,), rep1),
            pl.BlockSpec((3, _H2), rep),
            pl.BlockSpec((_H2, _H1), rep),
            pl.BlockSpec((_H1,), rep1),
            pl.BlockSpec((_H1, _DIN), rep),
            pl.BlockSpec((_DIN,), rep1),
        ],
        out_specs=[
            pl.BlockSpec((_BN2, _H2), row),
            pl.BlockSpec((_BN2, 2), row),
            pl.BlockSpec((_BN2, 3), row),
            pl.BlockSpec((_BN2, _H2), row),
            pl.BlockSpec((_BN2, _H2), row),
            pl.BlockSpec((_BN2, 1), row),
            pl.BlockSpec((_BN2, _DIN), row),
        ],
        out_shape=[
            jax.ShapeDtypeStruct((_N, _H2), jnp.float32),
            jax.ShapeDtypeStruct((_N, 2), jnp.float32),
            jax.ShapeDtypeStruct((_N, 3), jnp.float32),
            jax.ShapeDtypeStruct((_N, _H2), jnp.float32),
            jax.ShapeDtypeStruct((_N, _H2), jnp.float32),
            jax.ShapeDtypeStruct((_N, 1), jnp.float32),
            jax.ShapeDtypeStruct((_N, _DIN), jnp.float32),
        ],
    )(adj, adj2, p, w_omega, b_omega, u_omega, clusters, w_rec, b_rec,
      w_out, b_out)


def _decode_body(emb_ref, embt_ref, z1_ref, z2t_ref,
                 rec_ref, rowsum_ref, colsum_ref):
    i = pl.program_id(0)
    rec_ref[...] = _bf16dot(emb_ref[...], embt_ref[...])

    s = lax.dot(z1_ref[...].astype(jnp.bfloat16), z2t_ref[...],
                preferred_element_type=jnp.float32)
    es = jnp.exp(s)
    rowsum_ref[...] = jnp.sum(es, axis=1, keepdims=True)

    rid = i * _BND + lax.broadcasted_iota(jnp.int32, es.shape, 0)
    esm = jnp.where(rid < _N, es, 0.0)
    part = jnp.sum(esm, axis=0, keepdims=True)

    @pl.when(i == 0)
    def _():
        colsum_ref[...] = part

    @pl.when(i > 0)
    def _():
        colsum_ref[...] = colsum_ref[...] + part


def _decode_call(emb, embt, z1, z2t):
    grid = (pl.cdiv(_N, _BND),)
    row = lambda i: (i, 0)
    rep = lambda i: (0, 0)
    return pl.pallas_call(
        _decode_body,
        grid=grid,
        in_specs=[
            pl.BlockSpec((_BND, _H2), row),
            pl.BlockSpec((_H2, _N), rep),
            pl.BlockSpec((_BND, _H2), row),
            pl.BlockSpec((_H2, _N), rep),
        ],
        out_specs=[
            pl.BlockSpec((_BND, _N), row),
            pl.BlockSpec((_BND, 1), row),
            pl.BlockSpec((1, _N), rep),
        ],
        out_shape=[
            jax.ShapeDtypeStruct((_N, _N), jnp.float32),
            jax.ShapeDtypeStruct((_N, 1), jnp.float32),
            jax.ShapeDtypeStruct((1, _N), jnp.float32),
        ],
        compiler_params=pltpu.CompilerParams(
            dimension_semantics=("arbitrary",)),
    )(emb, embt, z1, z2t)


def kernel(features, adj, adj2, W1a, W1b, W2a, W2b, w_omega, b_omega,
           u_omega, clusters, W_rec, b_rec, W_out, b_out):
    wcat = jnp.concatenate([W1a, W1b], axis=1)
    xw = _xw_call(features, wcat)

    w2cat = jnp.concatenate([W2a, W2b], axis=1)
    p = _layer1_call(adj, adj2, xw, w2cat)

    emb, alphas, q, z1, z2, pos, rec3 = _layer2_call(
        adj, adj2, p, w_omega, b_omega, u_omega, clusters,
        W_rec, b_rec, W_out, b_out)

    rec, rowsum, colsum = _decode_call(
        emb, emb.T.astype(jnp.bfloat16), z1, z2.T.astype(jnp.bfloat16))

    cons_loss = jnp.mean(
        -pos[:, 0] + 0.5 * (jnp.log(rowsum[:, 0]) + jnp.log(colsum[0, :])))

    rec_flat = rec.reshape(-1)
    return (emb, alphas, q, cons_loss, rec_flat, rec_flat, rec3)


# speedup vs baseline: 1.3191x; 1.0393x over previous
import jax
import jax.numpy as jnp
from jax import lax
from jax.experimental import pallas as pl
from jax.experimental.pallas import tpu as pltpu

_N = 3025
_DIN = 1870
_H1 = 32
_H2 = 16

_BN1 = 1024
_BN2 = 512
_BND = 512


def _f32dot(a, b):
    return lax.dot(a, b, preferred_element_type=jnp.float32)


def _bf16dot(a, b):
    return lax.dot(a.astype(jnp.bfloat16), b.astype(jnp.bfloat16),
                   preferred_element_type=jnp.float32)


def _xw_body(x_ref, w_ref, o_ref):
    o_ref[...] = _bf16dot(x_ref[...], w_ref[...])


def _xw_call(features, wcat):
    grid = (pl.cdiv(_N, _BN1),)
    return pl.pallas_call(
        _xw_body,
        grid=grid,
        in_specs=[
            pl.BlockSpec((_BN1, _DIN), lambda i: (i, 0)),
            pl.BlockSpec((_DIN, 2 * _H1), lambda i: (0, 0)),
        ],
        out_specs=pl.BlockSpec((_BN1, 2 * _H1), lambda i: (i, 0)),
        out_shape=jax.ShapeDtypeStruct((_N, 2 * _H1), jnp.float32),
    )(features, wcat)


def _layer1_body(adj_ref, adj2_ref, xw_ref, w2_ref, p_ref):
    xw = xw_ref[...].astype(jnp.bfloat16)
    a1 = adj_ref[...].astype(jnp.bfloat16)
    a2 = adj2_ref[...].astype(jnp.bfloat16)
    h1 = jnp.maximum(lax.dot(a1, xw[:, :_H1], preferred_element_type=jnp.float32), 0.0)
    h2 = jnp.maximum(lax.dot(a2, xw[:, _H1:], preferred_element_type=jnp.float32), 0.0)
    w2 = w2_ref[...]
    p1 = _f32dot(h1, w2[:, :_H2])
    p2 = _f32dot(h2, w2[:, _H2:])
    p_ref[...] = jnp.concatenate([p1, p2], axis=1)


def _layer1_call(adj, adj2, xw, w2cat):
    grid = (pl.cdiv(_N, _BN2),)
    return pl.pallas_call(
        _layer1_body,
        grid=grid,
        in_specs=[
            pl.BlockSpec((_BN2, _N), lambda i: (i, 0)),
            pl.BlockSpec((_BN2, _N), lambda i: (i, 0)),
            pl.BlockSpec((_N, 2 * _H1), lambda i: (0, 0)),
            pl.BlockSpec((_H1, 2 * _H2), lambda i: (0, 0)),
        ],
        out_specs=pl.BlockSpec((_BN2, 2 * _H2), lambda i: (i, 0)),
        out_shape=jax.ShapeDtypeStruct((_N, 2 * _H2), jnp.float32),
    )(adj, adj2, xw, w2cat)


def _layer2_body(adj_ref, adj2_ref, p_ref, womega_ref, bomega_ref,
                 uomega_ref, clusters_ref, wrec_ref, brec_ref, wout_ref,
                 bout_ref,
                 emb_ref, al_ref, q_ref, z1_ref, z2_ref, pos_ref, rec3_ref):
    p = p_ref[...].astype(jnp.bfloat16)
    a1 = adj_ref[...].astype(jnp.bfloat16)
    a2 = adj2_ref[...].astype(jnp.bfloat16)
    e1 = lax.dot(a1, p[:, :_H2], preferred_element_type=jnp.float32)
    e2 = lax.dot(a2, p[:, _H2:], preferred_element_type=jnp.float32)

    w_om = womega_ref[...]
    b_om = bomega_ref[...][None, :]
    u_om = uomega_ref[...][None, :]
    v1 = jnp.tanh(_f32dot(e1, w_om) + b_om)
    v2 = jnp.tanh(_f32dot(e2, w_om) + b_om)
    vu1 = jnp.sum(v1 * u_om, axis=1, keepdims=True)
    vu2 = jnp.sum(v2 * u_om, axis=1, keepdims=True)
    m = jnp.maximum(vu1, vu2)
    x1 = jnp.exp(vu1 - m)
    x2 = jnp.exp(vu2 - m)
    s = x1 + x2
    al1 = x1 / s
    al2 = x2 / s
    emb = al1 * e1 + al2 * e2
    emb_ref[...] = emb
    al_ref[...] = jnp.concatenate([al1, al2], axis=1)

    c = clusters_ref[...]
    qs = []
    for k in range(3):
        ck = c[k:k + 1, :]
        d2 = jnp.sum((emb - ck) ** 2, axis=1, keepdims=True)
        qs.append(1.0 / (1.0 + d2))
    qcat = jnp.concatenate(qs, axis=1)
    q_ref[...] = qcat / jnp.sum(qcat, axis=1, keepdims=True)

    n1 = jnp.sqrt(jnp.sum(e1 * e1, axis=1, keepdims=True)) + 1e-8
    n2 = jnp.sqrt(jnp.sum(e2 * e2, axis=1, keepdims=True)) + 1e-8
    z1 = e1 / n1
    z2 = e2 / n2
    z1_ref[...] = z1
    z2_ref[...] = z2
    pos_ref[...] = jnp.sum(z1 * z2, axis=1, keepdims=True)

    r = jnp.maximum(_f32dot(emb, wrec_ref[...]) + brec_ref[...][None, :], 0.0)
    rec3_ref[...] = _bf16dot(r, wout_ref[...]) + bout_ref[...][None, :]


def _layer2_call(adj, adj2, p, w_omega, b_omega, u_omega, clusters,
                 w_rec, b_rec, w_out, b_out):
    grid = (pl.cdiv(_N, _BN2),)
    row = lambda i: (i, 0)
    rep = lambda i: (0, 0)
    rep1 = lambda i: (0,)
    return pl.pallas_call(
        _layer2_body,
        grid=grid,
        in_specs=[
            pl.BlockSpec((_BN2, _N), row),
            pl.BlockSpec((_BN2, _N), row),
            pl.BlockSpec((_N, 2 * _H2), rep),
            pl.BlockSpec((_H2, 16), rep),
            pl.BlockSpec((16,), rep1),
            pl.BlockSpec((16,), rep1),
            pl.BlockSpec((3, _H2), rep),
            pl.BlockSpec((_H2, _H1), rep),
            pl.BlockSpec((_H1,), rep1),
            pl.BlockSpec((_H1, _DIN), rep),
            pl.BlockSpec((_DIN,), rep1),
        ],
        out_specs=[
            pl.BlockSpec((_BN2, _H2), row),
            pl.BlockSpec((_BN2, 2), row),
            pl.BlockSpec((_BN2, 3), row),
            pl.BlockSpec((_BN2, _H2), row),
            pl.BlockSpec((_BN2, _H2), row),
            pl.BlockSpec((_BN2, 1), row),
            pl.BlockSpec((_BN2, _DIN), row),
        ],
        out_shape=[
            jax.ShapeDtypeStruct((_N, _H2), jnp.float32),
            jax.ShapeDtypeStruct((_N, 2), jnp.float32),
            jax.ShapeDtypeStruct((_N, 3), jnp.float32),
            jax.ShapeDtypeStruct((_N, _H2), jnp.float32),
            jax.ShapeDtypeStruct((_N, _H2), jnp.float32),
            jax.ShapeDtypeStruct((_N, 1), jnp.float32),
            jax.ShapeDtypeStruct((_N, _DIN), jnp.float32),
        ],
    )(adj, adj2, p, w_omega, b_omega, u_omega, clusters, w_rec, b_rec,
      w_out, b_out)


def _decode_body(emb_ref, embt_ref, z1_ref, z2t_ref,
                 rec_ref, rowsum_ref, colsum_ref):
    i = pl.program_id(0)
    rec_ref[...] = _bf16dot(emb_ref[...], embt_ref[...])

    s = lax.dot(z1_ref[...].astype(jnp.bfloat16), z2t_ref[...],
                preferred_element_type=jnp.float32)
    es = jnp.exp(s)
    rowsum_ref[...] = jnp.sum(es, axis=1, keepdims=True)

    rid = i * _BND + lax.broadcasted_iota(jnp.int32, es.shape, 0)
    esm = jnp.where(rid < _N, es, 0.0)
    part = jnp.sum(esm, axis=0, keepdims=True)

    @pl.when(i == 0)
    def _():
        colsum_ref[...] = part

    @pl.when(i > 0)
    def _():
        colsum_ref[...] = colsum_ref[...] + part


def _decode_call(emb, embt, z1, z2t):
    grid = (pl.cdiv(_N, _BND),)
    row = lambda i: (i, 0)
    rep = lambda i: (0, 0)
    return pl.pallas_call(
        _decode_body,
        grid=grid,
        in_specs=[
            pl.BlockSpec((_BND, _H2), row),
            pl.BlockSpec((_H2, _N), rep),
            pl.BlockSpec((_BND, _H2), row),
            pl.BlockSpec((_H2, _N), rep),
        ],
        out_specs=[
            pl.BlockSpec((_BND, _N), row),
            pl.BlockSpec((_BND, 1), row),
            pl.BlockSpec((1, _N), rep),
        ],
        out_shape=[
            jax.ShapeDtypeStruct((_N, _N), jnp.float32),
            jax.ShapeDtypeStruct((_N, 1), jnp.float32),
            jax.ShapeDtypeStruct((1, _N), jnp.float32),
        ],
        compiler_params=pltpu.CompilerParams(
            dimension_semantics=("arbitrary",)),
    )(emb, embt, z1, z2t)


def kernel(features, adj, adj2, W1a, W1b, W2a, W2b, w_omega, b_omega,
           u_omega, clusters, W_rec, b_rec, W_out, b_out):
    wcat = jnp.concatenate([W1a, W1b], axis=1)
    xw = _xw_call(features, wcat)

    w2cat = jnp.concatenate([W2a, W2b], axis=1)
    p = _layer1_call(adj, adj2, xw, w2cat)

    emb, alphas, q, z1, z2, pos, rec3 = _layer2_call(
        adj, adj2, p, w_omega, b_omega, u_omega, clusters,
        W_rec, b_rec, W_out, b_out)

    rec, rowsum, colsum = _decode_call(
        emb, emb.T.astype(jnp.bfloat16), z1, z2.T.astype(jnp.bfloat16))

    cons_loss = jnp.mean(
        -pos[:, 0] + 0.5 * (jnp.log(rowsum[:, 0]) + jnp.log(colsum[0, :])))

    rec_flat = rec.reshape(-1)
    return (emb, alphas, q, cons_loss, rec_flat, rec_flat, rec3)


# speedup vs baseline: 1.3628x; 1.0331x over previous
import jax
import jax.numpy as jnp
from jax import lax
from jax.experimental import pallas as pl
from jax.experimental.pallas import tpu as pltpu

_N = 3025
_DIN = 1870
_H1 = 32
_H2 = 16

_BN1 = 1024
_BN2 = 256
_BND = 512


def _f32dot(a, b):
    return lax.dot(a, b, preferred_element_type=jnp.float32)


def _bf16dot(a, b):
    return lax.dot(a.astype(jnp.bfloat16), b.astype(jnp.bfloat16),
                   preferred_element_type=jnp.float32)


def _xw_body(x_ref, w_ref, o_ref):
    o_ref[...] = _bf16dot(x_ref[...], w_ref[...])


def _xw_call(features, wcat):
    grid = (pl.cdiv(_N, _BN1),)
    return pl.pallas_call(
        _xw_body,
        grid=grid,
        in_specs=[
            pl.BlockSpec((_BN1, _DIN), lambda i: (i, 0)),
            pl.BlockSpec((_DIN, 2 * _H1), lambda i: (0, 0)),
        ],
        out_specs=pl.BlockSpec((_BN1, 2 * _H1), lambda i: (i, 0)),
        out_shape=jax.ShapeDtypeStruct((_N, 2 * _H1), jnp.float32),
    )(features, wcat)


def _gcn_body(adj_ref, adj2_ref, xw_ref, w2_ref, womega_ref, bomega_ref,
              uomega_ref, clusters_ref, wrec_ref, brec_ref, wout_ref,
              bout_ref,
              emb_ref, al_ref, q_ref, z1_ref, z2_ref, pos_ref, rec3_ref,
              adjbf_ref, adj2bf_ref, p_scr):
    ph = pl.program_id(0)
    i = pl.program_id(1)
    r0 = pl.multiple_of(i * _BN2, _BN2)

    @pl.when(ph == 0)
    def _():
        a1 = adj_ref[...].astype(jnp.bfloat16)
        a2 = adj2_ref[...].astype(jnp.bfloat16)
        adjbf_ref[pl.ds(r0, _BN2), :] = a1
        adj2bf_ref[pl.ds(r0, _BN2), :] = a2
        xw = xw_ref[...].astype(jnp.bfloat16)
        h1 = jnp.maximum(lax.dot(a1, xw[:, :_H1], preferred_element_type=jnp.float32), 0.0)
        h2 = jnp.maximum(lax.dot(a2, xw[:, _H1:], preferred_element_type=jnp.float32), 0.0)
        w2 = w2_ref[...]
        p_scr[pl.ds(r0, _BN2), :] = jnp.concatenate(
            [_f32dot(h1, w2[:, :_H2]), _f32dot(h2, w2[:, _H2:])], axis=1)

    @pl.when(ph == 1)
    def _():
        _head_block(adjbf_ref, adj2bf_ref, p_scr, r0, womega_ref,
                    bomega_ref, uomega_ref, clusters_ref, wrec_ref,
                    brec_ref, wout_ref, bout_ref, emb_ref, al_ref, q_ref,
                    z1_ref, z2_ref, pos_ref, rec3_ref)


def _head_block(adjbf_ref, adj2bf_ref, p_scr, r0, womega_ref, bomega_ref,
                uomega_ref, clusters_ref, wrec_ref, brec_ref, wout_ref,
                bout_ref,
                emb_ref, al_ref, q_ref, z1_ref, z2_ref, pos_ref, rec3_ref):
    a1 = adjbf_ref[pl.ds(r0, _BN2), :]
    a2 = adj2bf_ref[pl.ds(r0, _BN2), :]
    p = lax.slice(p_scr[...], (0, 0), (_N, 2 * _H2)).astype(jnp.bfloat16)
    e1 = lax.dot(a1, p[:, :_H2], preferred_element_type=jnp.float32)
    e2 = lax.dot(a2, p[:, _H2:], preferred_element_type=jnp.float32)

    w_om = womega_ref[...]
    b_om = bomega_ref[...][None, :]
    u_om = uomega_ref[...][None, :]
    v1 = jnp.tanh(_f32dot(e1, w_om) + b_om)
    v2 = jnp.tanh(_f32dot(e2, w_om) + b_om)
    vu1 = jnp.sum(v1 * u_om, axis=1, keepdims=True)
    vu2 = jnp.sum(v2 * u_om, axis=1, keepdims=True)
    m = jnp.maximum(vu1, vu2)
    x1 = jnp.exp(vu1 - m)
    x2 = jnp.exp(vu2 - m)
    s = x1 + x2
    al1 = x1 / s
    al2 = x2 / s
    emb = al1 * e1 + al2 * e2
    emb_ref[...] = emb
    al_ref[...] = jnp.concatenate([al1, al2], axis=1)

    c = clusters_ref[...]
    qs = []
    for k in range(3):
        ck = c[k:k + 1, :]
        d2 = jnp.sum((emb - ck) ** 2, axis=1, keepdims=True)
        qs.append(1.0 / (1.0 + d2))
    qcat = jnp.concatenate(qs, axis=1)
    q_ref[...] = qcat / jnp.sum(qcat, axis=1, keepdims=True)

    n1 = jnp.sqrt(jnp.sum(e1 * e1, axis=1, keepdims=True)) + 1e-8
    n2 = jnp.sqrt(jnp.sum(e2 * e2, axis=1, keepdims=True)) + 1e-8
    z1 = e1 / n1
    z2 = e2 / n2
    z1_ref[...] = z1
    z2_ref[...] = z2
    pos_ref[...] = jnp.sum(z1 * z2, axis=1, keepdims=True)

    r = jnp.maximum(_f32dot(emb, wrec_ref[...]) + brec_ref[...][None, :], 0.0)
    rec3_ref[...] = _bf16dot(r, wout_ref[...]) + bout_ref[...][None, :]


def _gcn_call(adj, adj2, xw, w2cat, w_omega, b_omega, u_omega, clusters,
              w_rec, b_rec, w_out, b_out):
    nb = pl.cdiv(_N, _BN2)
    grid = (2, nb)
    inrow = lambda ph, i: (i * (1 - ph), 0)
    outrow = lambda ph, i: (i * ph, 0)
    rep = lambda ph, i: (0, 0)
    rep1 = lambda ph, i: (0,)
    return pl.pallas_call(
        _gcn_body,
        grid=grid,
        in_specs=[
            pl.BlockSpec((_BN2, _N), inrow),
            pl.BlockSpec((_BN2, _N), inrow),
            pl.BlockSpec((_N, 2 * _H1), rep),
            pl.BlockSpec((_H1, 2 * _H2), rep),
            pl.BlockSpec((_H2, 16), rep),
            pl.BlockSpec((16,), rep1),
            pl.BlockSpec((16,), rep1),
            pl.BlockSpec((3, _H2), rep),
            pl.BlockSpec((_H2, _H1), rep),
            pl.BlockSpec((_H1,), rep1),
            pl.BlockSpec((_H1, _DIN), rep),
            pl.BlockSpec((_DIN,), rep1),
        ],
        out_specs=[
            pl.BlockSpec((_BN2, _H2), outrow),
            pl.BlockSpec((_BN2, 2), outrow),
            pl.BlockSpec((_BN2, 3), outrow),
            pl.BlockSpec((_BN2, _H2), outrow),
            pl.BlockSpec((_BN2, _H2), outrow),
            pl.BlockSpec((_BN2, 1), outrow),
            pl.BlockSpec((_BN2, _DIN), outrow),
        ],
        out_shape=[
            jax.ShapeDtypeStruct((_N, _H2), jnp.float32),
            jax.ShapeDtypeStruct((_N, 2), jnp.float32),
            jax.ShapeDtypeStruct((_N, 3), jnp.float32),
            jax.ShapeDtypeStruct((_N, _H2), jnp.float32),
            jax.ShapeDtypeStruct((_N, _H2), jnp.float32),
            jax.ShapeDtypeStruct((_N, 1), jnp.float32),
            jax.ShapeDtypeStruct((_N, _DIN), jnp.float32),
        ],
        scratch_shapes=[
            pltpu.VMEM((3072, _N), jnp.bfloat16),
            pltpu.VMEM((3072, _N), jnp.bfloat16),
            pltpu.VMEM((3072, 2 * _H2), jnp.float32),
        ],
        compiler_params=pltpu.CompilerParams(
            dimension_semantics=("arbitrary", "arbitrary")),
    )(adj, adj2, xw, w2cat, w_omega, b_omega, u_omega, clusters, w_rec,
      b_rec, w_out, b_out)


def _decode_body(emb_ref, embt_ref, z1_ref, z2t_ref,
                 rec_ref, rowsum_ref, colsum_ref):
    i = pl.program_id(0)
    rec_ref[...] = _bf16dot(emb_ref[...], embt_ref[...])

    s = lax.dot(z1_ref[...].astype(jnp.bfloat16), z2t_ref[...],
                preferred_element_type=jnp.float32)
    es = jnp.exp(s)
    rowsum_ref[...] = jnp.sum(es, axis=1, keepdims=True)

    rid = i * _BND + lax.broadcasted_iota(jnp.int32, es.shape, 0)
    esm = jnp.where(rid < _N, es, 0.0)
    part = jnp.sum(esm, axis=0, keepdims=True)

    @pl.when(i == 0)
    def _():
        colsum_ref[...] = part

    @pl.when(i > 0)
    def _():
        colsum_ref[...] = colsum_ref[...] + part


def _decode_call(emb, embt, z1, z2t):
    grid = (pl.cdiv(_N, _BND),)
    row = lambda i: (i, 0)
    rep = lambda i: (0, 0)
    return pl.pallas_call(
        _decode_body,
        grid=grid,
        in_specs=[
            pl.BlockSpec((_BND, _H2), row),
            pl.BlockSpec((_H2, _N), rep),
            pl.BlockSpec((_BND, _H2), row),
            pl.BlockSpec((_H2, _N), rep),
        ],
        out_specs=[
            pl.BlockSpec((_BND, _N), row),
            pl.BlockSpec((_BND, 1), row),
            pl.BlockSpec((1, _N), rep),
        ],
        out_shape=[
            jax.ShapeDtypeStruct((_N, _N), jnp.float32),
            jax.ShapeDtypeStruct((_N, 1), jnp.float32),
            jax.ShapeDtypeStruct((1, _N), jnp.float32),
        ],
        compiler_params=pltpu.CompilerParams(
            dimension_semantics=("arbitrary",)),
    )(emb, embt, z1, z2t)


def kernel(features, adj, adj2, W1a, W1b, W2a, W2b, w_omega, b_omega,
           u_omega, clusters, W_rec, b_rec, W_out, b_out):
    wcat = jnp.concatenate([W1a, W1b], axis=1)
    xw = _xw_call(features, wcat)

    w2cat = jnp.concatenate([W2a, W2b], axis=1)
    emb, alphas, q, z1, z2, pos, rec3 = _gcn_call(
        adj, adj2, xw, w2cat, w_omega, b_omega, u_omega, clusters,
        W_rec, b_rec, W_out, b_out)

    rec, rowsum, colsum = _decode_call(
        emb, emb.T.astype(jnp.bfloat16), z1, z2.T.astype(jnp.bfloat16))

    cons_loss = jnp.mean(
        -pos[:, 0] + 0.5 * (jnp.log(rowsum[:, 0]) + jnp.log(colsum[0, :])))

    rec_flat = rec.reshape(-1)
    return (emb, alphas, q, cons_loss, rec_flat, rec_flat, rec3)
